# baseline (speedup 1.0000x reference)
import functools

import jax
import jax.numpy as jnp
from jax import lax
from jax.experimental import pallas as pl
from jax.experimental.pallas import tpu as pltpu
from jax.experimental.pallas import tpu_sc as plsc

N = 10000
E = 320000
D = 128
H = 256
G = 64

NUM_TILES = 16
C = 125
NCHUNK = E // C
GB = 8
NPAD = 10240
RPT = NPAD // NUM_TILES
W = 128

_mesh = plsc.VectorSubcoreMesh(core_axis_name="c", subcore_axis_name="s")


def _acc_out(shape=(NPAD, W)):
    return [jax.ShapeDtypeStruct(shape, jnp.float32),
            jax.ShapeDtypeStruct(shape, jnp.float32)]


@functools.partial(
    pl.kernel,
    mesh=_mesh,
    out_type=_acc_out(),
    scratch_types=[
        pltpu.VMEM((NCHUNK // 32, C), jnp.int32),
        pltpu.VMEM((C, W), jnp.float32),
        pltpu.VMEM_SHARED((NPAD, W), jnp.float32),
        pltpu.SemaphoreType.DMA,
    ],
)
def _sc_count(dst2d_hbm, ones_hbm, zeros_hbm, out0_hbm, out1_hbm,
              dst2d_v, ones_v, acc_sh, sem):
    cid = lax.axis_index("c")
    sid = lax.axis_index("s")
    chunks = NCHUNK // 32
    base = cid * (NCHUNK // 2) + sid * chunks

    pltpu.sync_copy(zeros_hbm.at[pl.ds(sid * RPT, RPT)],
                    acc_sh.at[pl.ds(sid * RPT, RPT)])
    pltpu.sync_copy(dst2d_hbm.at[pl.ds(base, chunks)], dst2d_v)
    pltpu.sync_copy(ones_hbm, ones_v)
    plsc.subcore_barrier()

    def chunk(k, carry):
        pltpu.async_copy(ones_v, acc_sh.at[dst2d_v.at[k]], sem, add=True)

        @pl.when(k > 0)
        def _():
            pltpu.make_async_copy(ones_v, acc_sh.at[dst2d_v.at[0]], sem).wait()

        return carry

    lax.fori_loop(0, chunks, chunk, 0)
    pltpu.make_async_copy(ones_v, acc_sh.at[dst2d_v.at[0]], sem).wait()
    plsc.subcore_barrier()

    @pl.when(cid == 0)
    def _():
        pltpu.sync_copy(acc_sh.at[pl.ds(sid * RPT, RPT)],
                        out0_hbm.at[pl.ds(sid * RPT, RPT)])

    @pl.when(cid == 1)
    def _():
        pltpu.sync_copy(acc_sh.at[pl.ds(sid * RPT, RPT)],
                        out1_hbm.at[pl.ds(sid * RPT, RPT)])


def _make_sc_agg(batches, base_fn):
    @functools.partial(
        pl.kernel,
        mesh=_mesh,
        out_type=_acc_out(),
        scratch_types=[
            pltpu.VMEM((2, GB, C), jnp.int32),
            pltpu.VMEM((2, GB, C), jnp.int32),
            pltpu.VMEM((C, W), jnp.float32),
            pltpu.VMEM((C, W), jnp.float32),
            pltpu.VMEM_SHARED((NPAD, W), jnp.float32),
            pltpu.SemaphoreType.DMA,
            pltpu.SemaphoreType.DMA,
            pltpu.SemaphoreType.DMA,
            pltpu.SemaphoreType.DMA,
            pltpu.SemaphoreType.DMA,
            pltpu.SemaphoreType.DMA,
        ],
    )
    def sc_agg(h_hbm, comb_hbm, zeros_hbm, out0_hbm, out1_hbm,
               ib0, ib1, rows0_v, rows1_v, acc_sh,
               semi0, semi1, semg0, semg1, sems0, sems1):
        cid = lax.axis_index("c")
        sid = lax.axis_index("s")
        bbase = base_fn(cid, sid)
        rows = (rows0_v, rows1_v)
        semg = (semg0, semg1)
        sems = (sems0, sems1)

        pltpu.sync_copy(zeros_hbm.at[pl.ds(sid * RPT, RPT)],
                        acc_sh.at[pl.ds(sid * RPT, RPT)])
        plsc.subcore_barrier()

        pltpu.async_copy(comb_hbm.at[bbase], ib0, semi0)
        pltpu.async_copy(comb_hbm.at[bbase + 1], ib1, semi1)
        pltpu.make_async_copy(comb_hbm.at[bbase], ib0, semi0).wait()
        pltpu.async_copy(h_hbm.at[ib0.at[0, 0]], rows0_v, semg0)

        def batch_body(m, ib, ib_next, semi_self, semi_next):
            s_prev = None
            for k in range(GB):
                p = k % 2
                if k == 0:
                    pltpu.make_async_copy(h_hbm.at[ib.at[0, 0]], rows0_v,
                                          semg0).wait()
                else:
                    g_cur.wait()
                if s_prev is not None:
                    s_prev.wait()
                if k + 1 < GB:
                    g_cur = pltpu.async_copy(h_hbm.at[ib.at[0, k + 1]],
                                             rows[(k + 1) % 2],
                                             semg[(k + 1) % 2])
                else:
                    @pl.when(m + 1 < batches)
                    def _():
                        pltpu.make_async_copy(comb_hbm.at[bbase + m + 1],
                                              ib_next, semi_next).wait()
                        pltpu.async_copy(h_hbm.at[ib_next.at[0, 0]],
                                         rows0_v, semg0)
                s_prev = pltpu.async_copy(rows[p], acc_sh.at[ib.at[1, k]],
                                          sems[p], add=True)
            s_prev.wait()

            @pl.when(m + 2 < batches)
            def _():
                pltpu.async_copy(comb_hbm.at[bbase + m + 2], ib, semi_self)

        def outer(t, carry):
            batch_body(2 * t, ib0, ib1, semi0, semi1)
            batch_body(2 * t + 1, ib1, ib0, semi1, semi0)
            return carry

        lax.fori_loop(0, batches // 2, outer, 0)
        plsc.subcore_barrier()

        @pl.when(cid == 0)
        def _():
            pltpu.sync_copy(acc_sh.at[pl.ds(sid * RPT, RPT)],
                            out0_hbm.at[pl.ds(sid * RPT, RPT)])

        @pl.when(cid == 1)
        def _():
            pltpu.sync_copy(acc_sh.at[pl.ds(sid * RPT, RPT)],
                            out1_hbm.at[pl.ds(sid * RPT, RPT)])

    return sc_agg


assert GB % 2 == 0

_NB0 = NCHUNK // 32 // GB
_NB12 = NCHUNK // 16 // GB

_sc_agg_l0 = _make_sc_agg(
    _NB0, lambda cid, sid: cid * (16 * _NB0) + sid * _NB0)
_sc_agg_12 = _make_sc_agg(
    _NB12, lambda cid, sid: cid * (16 * _NB12) + sid * _NB12)


_B = 1000
_NB = N // _B


def _row_spec(w):
    return pl.BlockSpec((_B, w), lambda i: (i, 0))


def _full_spec(r, c):
    return pl.BlockSpec((r, c), lambda i: (0, 0))


_cnt_spec = pl.BlockSpec((_B, 1), lambda i: (i, 0))
_h3d_spec = pl.BlockSpec((2, _B, H // 2), lambda i: (0, i, 0))


def _tc_wr0_body(x, wrT, b, out):
    out[...] = (jnp.dot(x[...], wrT[...], preferred_element_type=jnp.float32)
                + b[...])


def _tc_wr12_body(h3d, wrT, b, out):
    h = jnp.concatenate([h3d[0], h3d[1]], axis=-1)
    out[...] = (jnp.dot(h, wrT[...], preferred_element_type=jnp.float32)
                + b[...])


def _tc_wr0(x, wrT, b):
    return pl.pallas_call(
        _tc_wr0_body,
        grid=(_NB,),
        in_specs=[_row_spec(D), _full_spec(D, H), _full_spec(1, H)],
        out_specs=_row_spec(H),
        out_shape=jax.ShapeDtypeStruct((N, H), jnp.float32),
    )(x, wrT, b)


def _tc_wr12(h3d, wrT, b):
    return pl.pallas_call(
        _tc_wr12_body,
        grid=(_NB,),
        in_specs=[_h3d_spec, _full_spec(H, H), _full_spec(1, H)],
        out_specs=_row_spec(H),
        out_shape=jax.ShapeDtypeStruct((N, H), jnp.float32),
    )(h3d, wrT, b)


def _tc_comb0_body(a0, a1, c0, c1, rw, wlT, g, be, out):
    _tc_comb_tail(a0[...] + a1[...], c0, c1, rw, wlT, g, be, out)


def _tc_comb12_body(a0, a1, c0, c1, rw, wlT, g, be, out):
    _tc_comb_tail(jnp.concatenate([a0[...], a1[...]], axis=-1),
                  c0, c1, rw, wlT, g, be, out)


def _tc_comb_tail(agg, c0, c1, rw, wlT, g, be, out):
    inv = 1.0 / jnp.maximum(c0[...] + c1[...], 1.0)
    m = (jnp.dot(agg * inv, wlT[...], preferred_element_type=jnp.float32)
         + rw[...])
    mu = jnp.mean(m, axis=-1, keepdims=True)
    xc = m - mu
    var = jnp.mean(xc * xc, axis=-1, keepdims=True)
    y = xc * lax.rsqrt(var + 1e-5) * g[...] + be[...]
    y = jnp.maximum(y, 0.0)
    out[0] = y[:, : H // 2]
    out[1] = y[:, H // 2:]


def _tc_comb(a0, a1, c0, c1, rw, wlT, g, be, Din, body):
    return pl.pallas_call(
        body,
        grid=(_NB,),
        in_specs=[
            _row_spec(Din // 2 if body is _tc_comb12_body else Din),
            _row_spec(Din // 2 if body is _tc_comb12_body else Din),
            _cnt_spec, _cnt_spec, _row_spec(H),
            _full_spec(Din, H), _full_spec(1, H), _full_spec(1, H),
        ],
        out_specs=_h3d_spec,
        out_shape=jax.ShapeDtypeStruct((2, N, H // 2), jnp.float32),
    )(a0, a1, c0, c1, rw, wlT, g, be)


def _tc_layer0(a0, a1, c0, c1, x, wlT, wrT, b, g, be):
    rw = _tc_wr0(x, wrT, b)
    return _tc_comb(a0, a1, c0, c1, rw, wlT, g, be, D, _tc_comb0_body)


def _tc_layer12(a0, a1, c0, c1, h3d, wlT, wrT, b, g, be):
    rw = _tc_wr12(h3d, wrT, b)
    return _tc_comb(a0, a1, c0, c1, rw, wlT, g, be, H, _tc_comb12_body)


def _tc_pool_body(h3d, batch, fcWT, fcb, out, accs, accc):
    i = pl.program_id(0)

    @pl.when(i == 0)
    def _():
        accs[...] = jnp.zeros_like(accs)
        accc[...] = jnp.zeros_like(accc)

    h = jnp.concatenate([h3d[0], h3d[1]], axis=-1)
    gids = lax.broadcasted_iota(jnp.int32, (1, G), 1)
    onehotT = (batch[...] == gids).astype(jnp.float32)
    accs[...] += lax.dot_general(onehotT, h, (((0,), (0,)), ((), ())),
                                 preferred_element_type=jnp.float32)
    ones = jnp.ones((_B, 128), jnp.float32)
    accc[...] += lax.dot_general(onehotT, ones, (((0,), (0,)), ((), ())),
                                 preferred_element_type=jnp.float32)

    @pl.when(i == _NB - 1)
    def _():
        pooled = accs[...] / jnp.maximum(accc[:, :1], 1.0)
        z = jnp.dot(pooled, fcWT[...], preferred_element_type=jnp.float32)
        out[...] = jnp.tanh(z + fcb[...])


def _tc_pool(h3d, batch2d, fcWT, fcb):
    return pl.pallas_call(
        _tc_pool_body,
        grid=(_NB,),
        in_specs=[
            _h3d_spec, _cnt_spec,
            _full_spec(H, H), _full_spec(1, H),
        ],
        out_specs=pl.BlockSpec((G, H), lambda i: (0, 0)),
        out_shape=jax.ShapeDtypeStruct((G, H), jnp.float32),
        scratch_shapes=[
            pltpu.VMEM((G, H), jnp.float32),
            pltpu.VMEM((G, 128), jnp.float32),
        ],
    )(h3d, batch2d, fcWT, fcb)


def kernel(x, edge_index, batch, Wl0, Wr0, b0, Wl1, Wr1, b1,
           Wl2, Wr2, b2, gamma, beta, fcW, fcb):
    src, dst = edge_index[0], edge_index[1]
    dst2d = dst.reshape(NCHUNK, C)
    srcA = src.reshape(NCHUNK // GB, GB, C)
    dstA = dst.reshape(NCHUNK // GB, GB, C)
    comb0 = jnp.stack([srcA, dstA], axis=1)
    comb12 = jnp.concatenate(
        [comb0, jnp.stack([srcA + N, dstA], axis=1)], axis=0)
    zeros_pad = jnp.zeros((NPAD, W), jnp.float32)
    ones_cw = jnp.ones((C, W), jnp.float32)
    batch2d = batch.reshape(N, 1)

    cnt0, cnt1 = _sc_count(dst2d, ones_cw, zeros_pad)
    c0 = cnt0[:, :1]
    c1 = cnt1[:, :1]

    b0r = b0.reshape(1, H)
    b1r = b1.reshape(1, H)
    b2r = b2.reshape(1, H)
    gr = gamma.reshape(1, H)
    ber = beta.reshape(1, H)
    fcbr = fcb.reshape(1, H)

    a00, a01 = _sc_agg_l0(x, comb0, zeros_pad)
    h1f = _tc_layer0(a00, a01, c0, c1, x, Wl0.T, Wr0.T, b0r, gr, ber)

    a10, a11 = _sc_agg_12(h1f.reshape(2 * N, H // 2), comb12, zeros_pad)
    h2f = _tc_layer12(a10, a11, c0, c1, h1f, Wl1.T, Wr1.T, b1r, gr, ber)

    a20, a21 = _sc_agg_12(h2f.reshape(2 * N, H // 2), comb12, zeros_pad)
    h3f = _tc_layer12(a20, a21, c0, c1, h2f, Wl2.T, Wr2.T, b2r, gr, ber)

    return _tc_pool(h3f, batch2d, fcW.T, fcbr)


# speedup vs baseline: 1.0934x; 1.0934x over previous
import functools

import jax
import jax.numpy as jnp
from jax import lax
from jax.experimental import pallas as pl
from jax.experimental.pallas import tpu as pltpu
from jax.experimental.pallas import tpu_sc as plsc

N = 10000
E = 320000
D = 128
H = 256
G = 64

NUM_TILES = 16
C = 125
NCHUNK = E // C
GB = 8
NPAD = 10240
RPT = NPAD // NUM_TILES
W = 128

_mesh = plsc.VectorSubcoreMesh(core_axis_name="c", subcore_axis_name="s")


def _acc_out(shape=(NPAD, W)):
    return [jax.ShapeDtypeStruct(shape, jnp.float32),
            jax.ShapeDtypeStruct(shape, jnp.float32)]


HR = NPAD // W

@functools.partial(
    pl.kernel,
    mesh=_mesh,
    out_type=_acc_out(shape=(HR, W)),
    compiler_params=pltpu.CompilerParams(needs_layout_passes=False),
    scratch_types=[
        pltpu.VMEM((HR, W), jnp.int32),
        pltpu.VMEM((HR, W), jnp.float32),
        pltpu.VMEM((1, HR), jnp.int32),
        pltpu.VMEM_SHARED((HR, W), jnp.float32),
        pltpu.SemaphoreType.DMA,
    ],
)
def _sc_count(dstp_hbm, iota_hbm, zeros_hbm, out0_hbm, out1_hbm,
              dst_v, hist_v, iota_v, acc_sh, sem):
    cid = lax.axis_index("c")
    sid = lax.axis_index("s")
    base = (cid * NUM_TILES + sid) * HR

    pltpu.sync_copy(dstp_hbm.at[pl.ds(base, HR)], dst_v)
    pltpu.sync_copy(zeros_hbm.at[pl.ds(0, HR)], hist_v)
    pltpu.sync_copy(iota_hbm, iota_v)

    @pl.when(sid == 0)
    def _():
        pltpu.sync_copy(zeros_hbm.at[pl.ds(0, HR)], acc_sh)

    plsc.subcore_barrier()

    ones16 = jnp.ones((16,), jnp.float32)

    def row(r, carry):
        for j in range(W // 16):
            d = dst_v[r, pl.ds(16 * j, 16)]
            plsc.addupdate_scatter(
                hist_v,
                [lax.shift_right_logical(d, 7), lax.bitwise_and(d, 127)],
                ones16)
        return carry

    lax.fori_loop(0, HR, row, 0)
    pltpu.async_copy(hist_v, acc_sh.at[iota_v.at[0]], sem, add=True)
    pltpu.make_async_copy(hist_v, acc_sh.at[iota_v.at[0]], sem).wait()
    plsc.subcore_barrier()

    @pl.when(jnp.logical_and(sid == 0, cid == 0))
    def _():
        pltpu.sync_copy(acc_sh, out0_hbm)

    @pl.when(jnp.logical_and(sid == 0, cid == 1))
    def _():
        pltpu.sync_copy(acc_sh, out1_hbm)


def _make_sc_agg(batches, base_fn):
    @functools.partial(
        pl.kernel,
        mesh=_mesh,
        out_type=_acc_out(),
        scratch_types=[
            pltpu.VMEM((2, GB, C), jnp.int32),
            pltpu.VMEM((2, GB, C), jnp.int32),
            pltpu.VMEM((C, W), jnp.float32),
            pltpu.VMEM((C, W), jnp.float32),
            pltpu.VMEM_SHARED((NPAD, W), jnp.float32),
            pltpu.SemaphoreType.DMA,
            pltpu.SemaphoreType.DMA,
            pltpu.SemaphoreType.DMA,
            pltpu.SemaphoreType.DMA,
            pltpu.SemaphoreType.DMA,
            pltpu.SemaphoreType.DMA,
        ],
    )
    def sc_agg(h_hbm, comb_hbm, zeros_hbm, out0_hbm, out1_hbm,
               ib0, ib1, rows0_v, rows1_v, acc_sh,
               semi0, semi1, semg0, semg1, sems0, sems1):
        cid = lax.axis_index("c")
        sid = lax.axis_index("s")
        bbase = base_fn(cid, sid)
        rows = (rows0_v, rows1_v)
        semg = (semg0, semg1)
        sems = (sems0, sems1)

        pltpu.sync_copy(zeros_hbm.at[pl.ds(sid * RPT, RPT)],
                        acc_sh.at[pl.ds(sid * RPT, RPT)])
        plsc.subcore_barrier()

        pltpu.async_copy(comb_hbm.at[bbase], ib0, semi0)
        pltpu.async_copy(comb_hbm.at[bbase + 1], ib1, semi1)
        pltpu.make_async_copy(comb_hbm.at[bbase], ib0, semi0).wait()
        pltpu.async_copy(h_hbm.at[ib0.at[0, 0]], rows0_v, semg0)

        def batch_body(m, ib, ib_next, semi_self, semi_next):
            s_prev = None
            for k in range(GB):
                p = k % 2
                if k == 0:
                    pltpu.make_async_copy(h_hbm.at[ib.at[0, 0]], rows0_v,
                                          semg0).wait()
                else:
                    g_cur.wait()
                if s_prev is not None:
                    s_prev.wait()
                if k + 1 < GB:
                    g_cur = pltpu.async_copy(h_hbm.at[ib.at[0, k + 1]],
                                             rows[(k + 1) % 2],
                                             semg[(k + 1) % 2])
                else:
                    @pl.when(m + 1 < batches)
                    def _():
                        pltpu.make_async_copy(comb_hbm.at[bbase + m + 1],
                                              ib_next, semi_next).wait()
                        pltpu.async_copy(h_hbm.at[ib_next.at[0, 0]],
                                         rows0_v, semg0)
                s_prev = pltpu.async_copy(rows[p], acc_sh.at[ib.at[1, k]],
                                          sems[p], add=True)
            s_prev.wait()

            @pl.when(m + 2 < batches)
            def _():
                pltpu.async_copy(comb_hbm.at[bbase + m + 2], ib, semi_self)

        def outer(t, carry):
            batch_body(2 * t, ib0, ib1, semi0, semi1)
            batch_body(2 * t + 1, ib1, ib0, semi1, semi0)
            return carry

        lax.fori_loop(0, batches // 2, outer, 0)
        plsc.subcore_barrier()

        @pl.when(cid == 0)
        def _():
            pltpu.sync_copy(acc_sh.at[pl.ds(sid * RPT, RPT)],
                            out0_hbm.at[pl.ds(sid * RPT, RPT)])

        @pl.when(cid == 1)
        def _():
            pltpu.sync_copy(acc_sh.at[pl.ds(sid * RPT, RPT)],
                            out1_hbm.at[pl.ds(sid * RPT, RPT)])

    return sc_agg


assert GB % 2 == 0

_NB0 = NCHUNK // 32 // GB
_NB12 = NCHUNK // 16 // GB

_sc_agg_l0 = _make_sc_agg(
    _NB0, lambda cid, sid: cid * (16 * _NB0) + sid * _NB0)
_sc_agg_12 = _make_sc_agg(
    _NB12, lambda cid, sid: cid * (16 * _NB12) + sid * _NB12)


_B = 1000
_NB = N // _B


def _row_spec(w):
    return pl.BlockSpec((_B, w), lambda i: (i, 0))


def _full_spec(r, c):
    return pl.BlockSpec((r, c), lambda i: (0, 0))


_cnt_spec = pl.BlockSpec((_B, 1), lambda i: (i, 0))
_h3d_spec = pl.BlockSpec((2, _B, H // 2), lambda i: (0, i, 0))


def _tc_layer0_body(a0, a1, cnt, x, wlT, wrT, b, g, be, out):
    rw = (jnp.dot(x[...], wrT[...], preferred_element_type=jnp.float32)
          + b[...])
    _tc_layer_tail(a0[...] + a1[...], cnt, rw, wlT, g, be, out)


def _tc_layer12_body(a0, a1, cnt, h3d, wlT, wrT, b, g, be, out):
    h = jnp.concatenate([h3d[0], h3d[1]], axis=-1)
    rw = (jnp.dot(h, wrT[...], preferred_element_type=jnp.float32)
          + b[...])
    _tc_layer_tail(jnp.concatenate([a0[...], a1[...]], axis=-1),
                   cnt, rw, wlT, g, be, out)


def _tc_layer_tail(agg, cnt, rw, wlT, g, be, out):
    inv = 1.0 / jnp.maximum(cnt[...], 1.0)
    m = (jnp.dot(agg * inv, wlT[...], preferred_element_type=jnp.float32)
         + rw)
    mu = jnp.mean(m, axis=-1, keepdims=True)
    xc = m - mu
    var = jnp.mean(xc * xc, axis=-1, keepdims=True)
    y = xc * lax.rsqrt(var + 1e-5) * g[...] + be[...]
    y = jnp.maximum(y, 0.0)
    out[0] = y[:, : H // 2]
    out[1] = y[:, H // 2:]


def _tc_layer0(a0, a1, cnt, x, wlT, wrT, b, g, be):
    return pl.pallas_call(
        _tc_layer0_body,
        grid=(_NB,),
        in_specs=[
            _row_spec(D), _row_spec(D), _cnt_spec, _row_spec(D),
            _full_spec(D, H), _full_spec(D, H),
            _full_spec(1, H), _full_spec(1, H), _full_spec(1, H),
        ],
        out_specs=_h3d_spec,
        out_shape=jax.ShapeDtypeStruct((2, N, H // 2), jnp.float32),
    )(a0, a1, cnt, x, wlT, wrT, b, g, be)


def _tc_layer12(a0, a1, cnt, h3d, wlT, wrT, b, g, be):
    return pl.pallas_call(
        _tc_layer12_body,
        grid=(_NB,),
        in_specs=[
            _row_spec(H // 2), _row_spec(H // 2), _cnt_spec, _h3d_spec,
            _full_spec(H, H), _full_spec(H, H),
            _full_spec(1, H), _full_spec(1, H), _full_spec(1, H),
        ],
        out_specs=_h3d_spec,
        out_shape=jax.ShapeDtypeStruct((2, N, H // 2), jnp.float32),
    )(a0, a1, cnt, h3d, wlT, wrT, b, g, be)


def _tc_pool_body(h3d, batch, fcWT, fcb, out, accs, accc):
    i = pl.program_id(0)

    @pl.when(i == 0)
    def _():
        accs[...] = jnp.zeros_like(accs)
        accc[...] = jnp.zeros_like(accc)

    h = jnp.concatenate([h3d[0], h3d[1]], axis=-1)
    gids = lax.broadcasted_iota(jnp.int32, (1, G), 1)
    onehotT = (batch[...] == gids).astype(jnp.float32)
    accs[...] += lax.dot_general(onehotT, h, (((0,), (0,)), ((), ())),
                                 preferred_element_type=jnp.float32)
    ones = jnp.ones((_B, 128), jnp.float32)
    accc[...] += lax.dot_general(onehotT, ones, (((0,), (0,)), ((), ())),
                                 preferred_element_type=jnp.float32)

    @pl.when(i == _NB - 1)
    def _():
        pooled = accs[...] / jnp.maximum(accc[:, :1], 1.0)
        z = jnp.dot(pooled, fcWT[...], preferred_element_type=jnp.float32)
        out[...] = jnp.tanh(z + fcb[...])


def _tc_pool(h3d, batch2d, fcWT, fcb):
    return pl.pallas_call(
        _tc_pool_body,
        grid=(_NB,),
        in_specs=[
            _h3d_spec, _cnt_spec,
            _full_spec(H, H), _full_spec(1, H),
        ],
        out_specs=pl.BlockSpec((G, H), lambda i: (0, 0)),
        out_shape=jax.ShapeDtypeStruct((G, H), jnp.float32),
        scratch_shapes=[
            pltpu.VMEM((G, H), jnp.float32),
            pltpu.VMEM((G, 128), jnp.float32),
        ],
    )(h3d, batch2d, fcWT, fcb)


def kernel(x, edge_index, batch, Wl0, Wr0, b0, Wl1, Wr1, b1,
           Wl2, Wr2, b2, gamma, beta, fcW, fcb):
    src, dst = edge_index[0], edge_index[1]
    srcA = src.reshape(NCHUNK // GB, GB, C)
    dstA = dst.reshape(NCHUNK // GB, GB, C)
    comb0 = jnp.stack([srcA, dstA], axis=1)
    comb12 = jnp.concatenate(
        [comb0, jnp.stack([srcA + N, dstA], axis=1)], axis=0)
    zeros_pad = jnp.zeros((NPAD, W), jnp.float32)
    batch2d = batch.reshape(N, 1)

    dstp = jnp.concatenate(
        [dst, jnp.full((2 * NUM_TILES * HR * W - E,), NPAD - 1, jnp.int32)]
    ).reshape(2 * NUM_TILES * HR, W)
    iota80 = jnp.arange(HR, dtype=jnp.int32).reshape(1, HR)

    cnt0, cnt1 = _sc_count(dstp, iota80, zeros_pad)
    cnt = (cnt0 + cnt1).reshape(NPAD, 1)

    b0r = b0.reshape(1, H)
    b1r = b1.reshape(1, H)
    b2r = b2.reshape(1, H)
    gr = gamma.reshape(1, H)
    ber = beta.reshape(1, H)
    fcbr = fcb.reshape(1, H)

    a00, a01 = _sc_agg_l0(x, comb0, zeros_pad)
    h1f = _tc_layer0(a00, a01, cnt, x, Wl0.T, Wr0.T, b0r, gr, ber)

    a10, a11 = _sc_agg_12(h1f.reshape(2 * N, H // 2), comb12, zeros_pad)
    h2f = _tc_layer12(a10, a11, cnt, h1f, Wl1.T, Wr1.T, b1r, gr, ber)

    a20, a21 = _sc_agg_12(h2f.reshape(2 * N, H // 2), comb12, zeros_pad)
    h3f = _tc_layer12(a20, a21, cnt, h2f, Wl2.T, Wr2.T, b2r, gr, ber)

    return _tc_pool(h3f, batch2d, fcW.T, fcbr)


# speedup vs baseline: 1.1022x; 1.0080x over previous
import functools

import jax
import jax.numpy as jnp
from jax import lax
from jax.experimental import pallas as pl
from jax.experimental.pallas import tpu as pltpu
from jax.experimental.pallas import tpu_sc as plsc

N = 10000
E = 320000
D = 128
H = 256
G = 64

NUM_TILES = 16
C = 125
NCHUNK = E // C
GB = 8
NPAD = 10240
RPT = NPAD // NUM_TILES
W = 128

_mesh = plsc.VectorSubcoreMesh(core_axis_name="c", subcore_axis_name="s")


def _acc_out(shape=(NPAD, W)):
    return [jax.ShapeDtypeStruct(shape, jnp.float32),
            jax.ShapeDtypeStruct(shape, jnp.float32)]


HR = NPAD // W

@functools.partial(
    pl.kernel,
    mesh=_mesh,
    out_type=_acc_out(shape=(HR, W)),
    compiler_params=pltpu.CompilerParams(needs_layout_passes=False),
    scratch_types=[
        pltpu.VMEM((HR, W), jnp.int32),
        pltpu.VMEM((HR, W), jnp.float32),
        pltpu.VMEM((1, HR), jnp.int32),
        pltpu.VMEM_SHARED((HR, W), jnp.float32),
        pltpu.SemaphoreType.DMA,
    ],
)
def _sc_count(dstp_hbm, iota_hbm, zeros_hbm, out0_hbm, out1_hbm,
              dst_v, hist_v, iota_v, acc_sh, sem):
    cid = lax.axis_index("c")
    sid = lax.axis_index("s")
    base = (cid * NUM_TILES + sid) * HR

    pltpu.sync_copy(dstp_hbm.at[pl.ds(base, HR)], dst_v)
    pltpu.sync_copy(zeros_hbm.at[pl.ds(0, HR)], hist_v)
    pltpu.sync_copy(iota_hbm, iota_v)

    @pl.when(sid == 0)
    def _():
        pltpu.sync_copy(zeros_hbm.at[pl.ds(0, HR)], acc_sh)

    plsc.subcore_barrier()

    ones16 = jnp.ones((16,), jnp.float32)

    def row(r, carry):
        for j in range(W // 16):
            d = dst_v[r, pl.ds(16 * j, 16)]
            plsc.addupdate_scatter(
                hist_v,
                [lax.shift_right_logical(d, 7), lax.bitwise_and(d, 127)],
                ones16)
        return carry

    lax.fori_loop(0, HR, row, 0)
    pltpu.async_copy(hist_v, acc_sh.at[iota_v.at[0]], sem, add=True)
    pltpu.make_async_copy(hist_v, acc_sh.at[iota_v.at[0]], sem).wait()
    plsc.subcore_barrier()

    @pl.when(jnp.logical_and(sid == 0, cid == 0))
    def _():
        pltpu.sync_copy(acc_sh, out0_hbm)

    @pl.when(jnp.logical_and(sid == 0, cid == 1))
    def _():
        pltpu.sync_copy(acc_sh, out1_hbm)


def _make_sc_agg(batches, base_fn):
    @functools.partial(
        pl.kernel,
        mesh=_mesh,
        out_type=_acc_out(),
        scratch_types=[
            pltpu.VMEM((2, GB, C), jnp.int32),
            pltpu.VMEM((2, GB, C), jnp.int32),
            pltpu.VMEM((C, W), jnp.float32),
            pltpu.VMEM((C, W), jnp.float32),
            pltpu.VMEM_SHARED((NPAD, W), jnp.float32),
            pltpu.SemaphoreType.DMA,
            pltpu.SemaphoreType.DMA,
            pltpu.SemaphoreType.DMA,
            pltpu.SemaphoreType.DMA,
            pltpu.SemaphoreType.DMA,
            pltpu.SemaphoreType.DMA,
        ],
    )
    def sc_agg(h_hbm, comb_hbm, zeros_hbm, out0_hbm, out1_hbm,
               ib0, ib1, rows0_v, rows1_v, acc_sh,
               semi0, semi1, semg0, semg1, sems0, sems1):
        cid = lax.axis_index("c")
        sid = lax.axis_index("s")
        bbase = base_fn(cid, sid)
        rows = (rows0_v, rows1_v)
        semg = (semg0, semg1)
        sems = (sems0, sems1)

        pltpu.sync_copy(zeros_hbm.at[pl.ds(sid * RPT, RPT)],
                        acc_sh.at[pl.ds(sid * RPT, RPT)])
        plsc.subcore_barrier()

        pltpu.async_copy(comb_hbm.at[bbase], ib0, semi0)
        pltpu.async_copy(comb_hbm.at[bbase + 1], ib1, semi1)
        pltpu.make_async_copy(comb_hbm.at[bbase], ib0, semi0).wait()
        pltpu.async_copy(h_hbm.at[ib0.at[0, 0]], rows0_v, semg0)

        def batch_body(m, ib, ib_next, semi_self, semi_next):
            s_prev = None
            for k in range(GB):
                p = k % 2
                if k == 0:
                    pltpu.make_async_copy(h_hbm.at[ib.at[0, 0]], rows0_v,
                                          semg0).wait()
                else:
                    g_cur.wait()
                if s_prev is not None:
                    s_prev.wait()
                if k + 1 < GB:
                    g_cur = pltpu.async_copy(h_hbm.at[ib.at[0, k + 1]],
                                             rows[(k + 1) % 2],
                                             semg[(k + 1) % 2])
                else:
                    @pl.when(m + 1 < batches)
                    def _():
                        pltpu.make_async_copy(comb_hbm.at[bbase + m + 1],
                                              ib_next, semi_next).wait()
                        pltpu.async_copy(h_hbm.at[ib_next.at[0, 0]],
                                         rows0_v, semg0)
                s_prev = pltpu.async_copy(rows[p], acc_sh.at[ib.at[1, k]],
                                          sems[p], add=True)
            s_prev.wait()

            @pl.when(m + 2 < batches)
            def _():
                pltpu.async_copy(comb_hbm.at[bbase + m + 2], ib, semi_self)

        def outer(t, carry):
            batch_body(2 * t, ib0, ib1, semi0, semi1)
            batch_body(2 * t + 1, ib1, ib0, semi1, semi0)
            return carry

        lax.fori_loop(0, batches // 2, outer, 0)
        plsc.subcore_barrier()

        @pl.when(cid == 0)
        def _():
            pltpu.sync_copy(acc_sh.at[pl.ds(sid * RPT, RPT)],
                            out0_hbm.at[pl.ds(sid * RPT, RPT)])

        @pl.when(cid == 1)
        def _():
            pltpu.sync_copy(acc_sh.at[pl.ds(sid * RPT, RPT)],
                            out1_hbm.at[pl.ds(sid * RPT, RPT)])

    return sc_agg


assert GB % 2 == 0

_NB0 = NCHUNK // 32 // GB
_NB12 = NCHUNK // 16 // GB

_sc_agg_l0 = _make_sc_agg(
    _NB0, lambda cid, sid: cid * (16 * _NB0) + sid * _NB0)
_sc_agg_12 = _make_sc_agg(
    _NB12, lambda cid, sid: cid * (16 * _NB12) + sid * _NB12)


_B = 1000
_NB = N // _B


def _row_spec(w):
    return pl.BlockSpec((_B, w), lambda i: (i, 0))


def _full_spec(r, c):
    return pl.BlockSpec((r, c), lambda i: (0, 0))


_cnt_spec = pl.BlockSpec((_B, 1), lambda i: (i, 0))
_h3d_spec = pl.BlockSpec((2, _B, H // 2), lambda i: (0, i, 0))


def _tc_layer0_body(a0, a1, cnt, x, wlT, wrT, b, g, be, out):
    rw = (jnp.dot(x[...], wrT[...], preferred_element_type=jnp.float32)
          + b[...])
    y = _tc_layer_tail(a0[...] + a1[...], cnt, rw, wlT, g, be)
    out[0] = y[:, : H // 2]
    out[1] = y[:, H // 2:]


def _tc_layer12_body(a0, a1, cnt, h3d, wlT, wrT, b, g, be, out):
    h = jnp.concatenate([h3d[0], h3d[1]], axis=-1)
    rw = (jnp.dot(h, wrT[...], preferred_element_type=jnp.float32)
          + b[...])
    y = _tc_layer_tail(jnp.concatenate([a0[...], a1[...]], axis=-1),
                       cnt, rw, wlT, g, be)
    out[0] = y[:, : H // 2]
    out[1] = y[:, H // 2:]


def _tc_layer_tail(agg, cnt, rw, wlT, g, be):
    inv = 1.0 / jnp.maximum(cnt[...], 1.0)
    m = (jnp.dot(agg * inv, wlT[...], preferred_element_type=jnp.float32)
         + rw)
    mu = jnp.mean(m, axis=-1, keepdims=True)
    xc = m - mu
    var = jnp.mean(xc * xc, axis=-1, keepdims=True)
    y = xc * lax.rsqrt(var + 1e-5) * g[...] + be[...]
    return jnp.maximum(y, 0.0)


def _tc_layer2_pool_body(a0, a1, cnt, h3d, wlT, wrT, b, g, be,
                         batch, fcWT, fcb, out, accs, accc):
    i = pl.program_id(0)

    @pl.when(i == 0)
    def _():
        accs[...] = jnp.zeros_like(accs)
        accc[...] = jnp.zeros_like(accc)

    h = jnp.concatenate([h3d[0], h3d[1]], axis=-1)
    rw = (jnp.dot(h, wrT[...], preferred_element_type=jnp.float32)
          + b[...])
    y = _tc_layer_tail(jnp.concatenate([a0[...], a1[...]], axis=-1),
                       cnt, rw, wlT, g, be)
    gids = lax.broadcasted_iota(jnp.int32, (1, G), 1)
    onehotT = (batch[...] == gids).astype(jnp.float32)
    accs[...] += lax.dot_general(onehotT, y, (((0,), (0,)), ((), ())),
                                 preferred_element_type=jnp.float32)
    ones = jnp.ones((_B, 128), jnp.float32)
    accc[...] += lax.dot_general(onehotT, ones, (((0,), (0,)), ((), ())),
                                 preferred_element_type=jnp.float32)

    @pl.when(i == _NB - 1)
    def _():
        pooled = accs[...] / jnp.maximum(accc[:, :1], 1.0)
        z = jnp.dot(pooled, fcWT[...], preferred_element_type=jnp.float32)
        out[...] = jnp.tanh(z + fcb[...])


def _tc_layer0(a0, a1, cnt, x, wlT, wrT, b, g, be):
    return pl.pallas_call(
        _tc_layer0_body,
        grid=(_NB,),
        in_specs=[
            _row_spec(D), _row_spec(D), _cnt_spec, _row_spec(D),
            _full_spec(D, H), _full_spec(D, H),
            _full_spec(1, H), _full_spec(1, H), _full_spec(1, H),
        ],
        out_specs=_h3d_spec,
        out_shape=jax.ShapeDtypeStruct((2, N, H // 2), jnp.float32),
    )(a0, a1, cnt, x, wlT, wrT, b, g, be)


def _tc_layer12(a0, a1, cnt, h3d, wlT, wrT, b, g, be):
    return pl.pallas_call(
        _tc_layer12_body,
        grid=(_NB,),
        in_specs=[
            _row_spec(H // 2), _row_spec(H // 2), _cnt_spec, _h3d_spec,
            _full_spec(H, H), _full_spec(H, H),
            _full_spec(1, H), _full_spec(1, H), _full_spec(1, H),
        ],
        out_specs=_h3d_spec,
        out_shape=jax.ShapeDtypeStruct((2, N, H // 2), jnp.float32),
    )(a0, a1, cnt, h3d, wlT, wrT, b, g, be)


def _tc_layer2_pool(a0, a1, cnt, h3d, wlT, wrT, b, g, be,
                    batch2d, fcWT, fcb):
    return pl.pallas_call(
        _tc_layer2_pool_body,
        grid=(_NB,),
        in_specs=[
            _row_spec(H // 2), _row_spec(H // 2), _cnt_spec, _h3d_spec,
            _full_spec(H, H), _full_spec(H, H),
            _full_spec(1, H), _full_spec(1, H), _full_spec(1, H),
            _cnt_spec, _full_spec(H, H), _full_spec(1, H),
        ],
        out_specs=pl.BlockSpec((G, H), lambda i: (0, 0)),
        out_shape=jax.ShapeDtypeStruct((G, H), jnp.float32),
        scratch_shapes=[
            pltpu.VMEM((G, H), jnp.float32),
            pltpu.VMEM((G, 128), jnp.float32),
        ],
    )(a0, a1, cnt, h3d, wlT, wrT, b, g, be, batch2d, fcWT, fcb)


def kernel(x, edge_index, batch, Wl0, Wr0, b0, Wl1, Wr1, b1,
           Wl2, Wr2, b2, gamma, beta, fcW, fcb):
    src, dst = edge_index[0], edge_index[1]
    srcA = src.reshape(NCHUNK // GB, GB, C)
    dstA = dst.reshape(NCHUNK // GB, GB, C)
    comb0 = jnp.stack([srcA, dstA], axis=1)
    comb12 = jnp.concatenate(
        [comb0, jnp.stack([srcA + N, dstA], axis=1)], axis=0)
    zeros_pad = jnp.zeros((NPAD, W), jnp.float32)
    batch2d = batch.reshape(N, 1)

    dstp = jnp.concatenate(
        [dst, jnp.full((2 * NUM_TILES * HR * W - E,), NPAD - 1, jnp.int32)]
    ).reshape(2 * NUM_TILES * HR, W)
    iota80 = jnp.arange(HR, dtype=jnp.int32).reshape(1, HR)

    cnt0, cnt1 = _sc_count(dstp, iota80, zeros_pad)
    cnt = (cnt0 + cnt1).reshape(NPAD, 1)

    b0r = b0.reshape(1, H)
    b1r = b1.reshape(1, H)
    b2r = b2.reshape(1, H)
    gr = gamma.reshape(1, H)
    ber = beta.reshape(1, H)
    fcbr = fcb.reshape(1, H)

    a00, a01 = _sc_agg_l0(x, comb0, zeros_pad)
    h1f = _tc_layer0(a00, a01, cnt, x, Wl0.T, Wr0.T, b0r, gr, ber)

    a10, a11 = _sc_agg_12(h1f.reshape(2 * N, H // 2), comb12, zeros_pad)
    h2f = _tc_layer12(a10, a11, cnt, h1f, Wl1.T, Wr1.T, b1r, gr, ber)

    a20, a21 = _sc_agg_12(h2f.reshape(2 * N, H // 2), comb12, zeros_pad)
    return _tc_layer2_pool(a20, a21, cnt, h2f, Wl2.T, Wr2.T, b2r, gr, ber,
                           batch2d, fcW.T, fcbr)
